# baseline (speedup 1.0000x reference)
import functools

import jax
import jax.numpy as jnp
from jax import lax
from jax.experimental import pallas as pl
from jax.experimental.pallas import tpu as pltpu
from jax.experimental.pallas import tpu_sc as plsc

_R = 8192
_S = 64
_C = 32
_P = _R * _S
_GAIN = 1.0
_NW = 32
_PW = _P // _NW
_CH = 128


_GDN = lax.GatherDimensionNumbers(offset_dims=(), collapsed_slice_dims=(0,),
                                  start_index_map=(0,))


def _lane_bcast(v, jv):
    return lax.gather(v, jv[:, None], _GDN, (1,),
                      mode=lax.GatherScatterMode.PROMISE_IN_BOUNDS)


def _compute_block(gb, wt_s, out_b):
    for g16 in range(_CH // 16):
        ws = []
        for t in range(3):
            tx = wt_s[2 * t, pl.ds(g16 * 16, 16)]
            ty = wt_s[2 * t + 1, pl.ds(g16 * 16, 16)]
            ws += [(1.0 - tx) * (1.0 - ty), tx * (1.0 - ty),
                   (1.0 - tx) * ty, tx * ty]

        hi_mask = jnp.full((16,), -65536, jnp.int32)
        rnd = jnp.full((16,), 32768, jnp.int32)

        def pt_body(jj, c2, g16=g16, ws=ws):
            j = g16 * 16 + jj
            jv = jnp.zeros((16,), jnp.int32) + jj
            wb = [_lane_bcast(w, jv) for w in ws]
            for g in range(2):
                acc_a = None
                acc_b = None
                for t in range(3):
                    for q in range(4):
                        v = gb[t][j, pl.ds(32 * q + 16 * g, 16)]
                        va = lax.bitcast_convert_type(v << 16, jnp.float32)
                        vb = lax.bitcast_convert_type(v & hi_mask, jnp.float32)
                        w = wb[4 * t + q]
                        if acc_a is None:
                            acc_a, acc_b = w * va, w * vb
                        else:
                            acc_a = acc_a + w * va
                            acc_b = acc_b + w * vb
                ia = lax.bitcast_convert_type(acc_a, jnp.int32) + rnd
                ib = lax.bitcast_convert_type(acc_b, jnp.int32) + rnd
                out_b[j, pl.ds(16 * g, 16)] = (
                    lax.shift_right_logical(ia, 16) | (ib & hi_mask))
            return c2

        lax.fori_loop(0, 16, pt_body, 0)


def _sc_gather(t_xy, t_yz, t_zx, idx_all, wt_all):
    mesh = plsc.VectorSubcoreMesh(core_axis_name="c", subcore_axis_name="s")
    nb = _PW // _CH

    @functools.partial(
        pl.kernel,
        mesh=mesh,
        out_type=jax.ShapeDtypeStruct((_P, 32), jnp.int32),
        scratch_types=[
            *[pltpu.VMEM((3, _CH), jnp.int32) for _ in range(2)],
            *[pltpu.VMEM((6, _CH), jnp.float32) for _ in range(2)],
            *[pltpu.VMEM((_CH, 128), jnp.int32) for _ in range(6)],
            pltpu.VMEM((_CH, 32), jnp.int32),
            *[pltpu.SemaphoreType.DMA for _ in range(5)],
        ],
    )
    def k(txy, tyz, tzx, idx_hbm, wt_hbm, out_hbm,
          i0, i1, w0, w1,
          ga0, ga1, ga2, gb0, gb1, gb2,
          outb, gsem0, gsem1, isem0, isem1, osem):
        tabs = (txy, tyz, tzx)
        idx_s = (i0, i1)
        wt_s = (w0, w1)
        gb = ((ga0, ga1, ga2), (gb0, gb1, gb2))
        gsem = (gsem0, gsem1)
        isem = (isem0, isem1)
        wid = lax.axis_index("s") * 2 + lax.axis_index("c")
        w_base = wid * _PW

        def fire_idx(b, p):
            pltpu.async_copy(idx_hbm.at[:, pl.ds(w_base + b * _CH, _CH)],
                             idx_s[p], isem[p])
            pltpu.async_copy(wt_hbm.at[:, pl.ds(w_base + b * _CH, _CH)],
                             wt_s[p], isem[p])

        def wait_idx(p):
            pltpu.make_async_copy(idx_hbm.at[:, pl.ds(0, _CH)],
                                  idx_s[p], isem[p]).wait()
            pltpu.make_async_copy(wt_hbm.at[:, pl.ds(0, _CH)],
                                  wt_s[p], isem[p]).wait()

        def fire_gathers(p):
            for t in range(3):
                pltpu.async_copy(tabs[t].at[idx_s[p].at[t]],
                                 gb[p][t], gsem[p])

        def wait_gathers(p):
            for t in range(3):
                pltpu.make_async_copy(tabs[t].at[idx_s[p].at[t]],
                                      gb[p][t], gsem[p]).wait()

        def wait_out():
            pltpu.make_async_copy(outb, out_hbm.at[pl.ds(w_base, _CH)],
                                  osem).wait()

        fire_idx(0, 0)
        wait_idx(0)
        fire_gathers(0)
        fire_idx(1, 1)

        def loop_body(bb, carry):
            for p in range(2):
                b = bb * 2 + p
                q = 1 - p
                wait_idx(q)
                fire_gathers(q)
                wait_gathers(p)
                if p == 0:
                    @pl.when(bb >= 1)
                    def _w():
                        wait_out()
                else:
                    wait_out()
                _compute_block(gb[p], wt_s[p], outb)
                pltpu.async_copy(
                    outb, out_hbm.at[pl.ds(w_base + b * _CH, _CH)], osem)
                bn = jnp.minimum(b + 2, nb - 1)
                fire_idx(bn, p)
            return carry

        lax.fori_loop(0, nb // 2, loop_body, 0)

        wait_idx(1)
        wait_gathers(0)
        wait_out()

    return k(t_xy, t_yz, t_zx, idx_all, wt_all)


def _softplus(x):
    return jnp.maximum(x, 0.0) + jnp.log1p(jnp.exp(-jnp.abs(x)))


def _tc_body(x_ref, enc_ref, nf_ref, wb_ref, w1t_ref, b0_ref, b1_ref,
             wop_ref, bop_ref, wct_ref, bc_ref, bg_ref,
             out_ref, nlt_ref, awhc_ref, aw_ref, awt_ref):
    s = pl.program_id(0)

    @pl.when(s == 0)
    def _init():
        nlt_ref[...] = jnp.zeros_like(nlt_ref)
        awhc_ref[...] = jnp.zeros_like(awhc_ref)
        aw_ref[...] = jnp.zeros_like(aw_ref)
        awt_ref[...] = jnp.zeros_like(awt_ref)

    x = x_ref[...].astype(jnp.float32)
    z = lax.dot_general(wb_ref[...], x, (((1,), (1,)), ((), ())),
                        preferred_element_type=jnp.float32)
    h = _softplus(z[0:_C, :] + b0_ref[...])
    h = _softplus(jnp.dot(w1t_ref[...], h,
                          preferred_element_type=jnp.float32) + b1_ref[...])
    sig_pre = jnp.sum(h * wop_ref[...], axis=0, keepdims=True) + bop_ref[...]
    sigma = _softplus(sig_pre)

    near = nf_ref[0:1, :]
    far = nf_ref[1:2, :]
    sf = (s.astype(jnp.float32) + 0.5) * (1.0 / _S)
    t = near + (far - near) * sf
    delta = (far - near) * (1.0 / _S)
    sd = sigma * delta

    nlt0 = nlt_ref[...]
    wgt = jnp.exp(-nlt0) * (1.0 - jnp.exp(-sd))
    nlt_ref[...] = nlt0 + sd

    hc = _softplus(h + z[_C:2 * _C, :] + enc_ref[...])
    awhc_ref[...] += wgt * hc
    aw_ref[...] += wgt
    awt_ref[...] += wgt * t

    @pl.when(s == _S - 1)
    def _fin():
        mask = 1.0 - jnp.exp(-nlt_ref[...])
        cf = (jnp.dot(wct_ref[...], awhc_ref[...],
                      preferred_element_type=jnp.float32)
              + bc_ref[...] * aw_ref[...])
        fr = cf[0:3, :] + (1.0 - mask) * bg_ref[...]
        out_ref[...] = jnp.concatenate(
            [fr, mask, awt_ref[...], jnp.zeros((3, _R), jnp.float32)], axis=0)


def _render(x, enc_t, nf, wb, w1t, b0, b1, wop, bop, wct, bc, bg):
    full = lambda shape: pl.BlockSpec(shape, lambda s: (0, 0))
    return pl.pallas_call(
        _tc_body,
        grid=(_S,),
        in_specs=[
            pl.BlockSpec((_R, 64), lambda s: (s, 0)),
            full((_C, _R)), full((2, _R)), full((64, 64)), full((_C, _C)),
            full((_C, 1)), full((_C, 1)), full((_C, 1)), full((1, 1)),
            full((16, _C)), full((16, 1)), full((3, 1)),
        ],
        out_specs=full((8, _R)),
        out_shape=jax.ShapeDtypeStruct((8, _R), jnp.float32),
        scratch_shapes=[
            pltpu.VMEM((1, _R), jnp.float32),
            pltpu.VMEM((_C, _R), jnp.float32),
            pltpu.VMEM((1, _R), jnp.float32),
            pltpu.VMEM((1, _R), jnp.float32),
        ],
        compiler_params=pltpu.CompilerParams(
            dimension_semantics=("arbitrary",)),
    )(x, enc_t, nf, wb, w1t, b0, b1, wop, bop, wct, bc, bg)


def _corner_idx(u, v):
    x = jnp.clip((u + 1.0) * (0.5 * 255.0), 0.0, 255.0)
    y = jnp.clip((v + 1.0) * (0.5 * 255.0), 0.0, 255.0)
    x0 = jnp.floor(x)
    y0 = jnp.floor(y)
    tx = x - x0
    ty = y - y0
    i00 = y0.astype(jnp.int32) * 256 + x0.astype(jnp.int32)
    return i00, tx, ty


def _shift_rows(a, n):
    return jnp.concatenate([a[n:], a[-n:]], axis=0)


def _pack_table(feat, color):
    base = jnp.concatenate([feat, color], axis=0).transpose(1, 2, 0).reshape(
        256 * 256, 64).astype(jnp.bfloat16)
    s1 = _shift_rows(base, 1)
    s256 = _shift_rows(base, 256)
    s257 = _shift_rows(s1, 256)
    quad = jnp.concatenate([base, s1, s256, s257], axis=1)
    return jax.lax.bitcast_convert_type(
        quad.reshape(256 * 256, 128, 2), jnp.int32)


def kernel(rays, centers, rays_encoding, near, far, xy, yz, zx,
           xy_color, yz_color, zx_color, mlp_weights, mlp_biases,
           weight_opacity, bias_opacity, weight_color, bias_color, bg_color):
    sgrid = (jnp.arange(_S, dtype=jnp.float32) + 0.5) / _S
    t = near[None, :] + (far - near)[None, :] * sgrid[:, None]
    pts = centers[None, :, :] + t[..., None] * rays[None, :, :]
    px = pts[..., 0].reshape(_P)
    py = pts[..., 1].reshape(_P)
    pz = pts[..., 2].reshape(_P)

    idx_rows = []
    wt_rows = []
    for u, v in ((px, py), (py, pz), (pz, px)):
        i00, tx, ty = _corner_idx(u, v)
        idx_rows.append(i00)
        wt_rows += [tx, ty]
    idx_all = jnp.stack(idx_rows)
    wt_all = jnp.stack(wt_rows)

    t_xy = _pack_table(xy, xy_color)
    t_yz = _pack_table(yz, yz_color)
    t_zx = _pack_table(zx, zx_color)

    x_i32 = _sc_gather(t_xy, t_yz, t_zx, idx_all, wt_all)
    x = jax.lax.bitcast_convert_type(x_i32, jnp.bfloat16).reshape(_P, 64)

    zero = jnp.zeros((_C, _C), jnp.float32)
    wb = jnp.concatenate([
        jnp.concatenate([_GAIN * mlp_weights[0].T, zero], axis=1),
        jnp.concatenate([zero, _GAIN * jnp.eye(_C, dtype=jnp.float32)], axis=1),
    ], axis=0)

    out = _render(
        x, rays_encoding.T, jnp.stack([near, far]), wb,
        mlp_weights[1].T, mlp_biases[0][:, None], mlp_biases[1][:, None],
        weight_opacity[:, None], bias_opacity.reshape(1, 1),
        weight_color.T, bias_color[:, None], bg_color[:, None])

    feature_render = out[0:3, :].T
    mask = out[3, :]
    ray_length = out[4, :]
    return feature_render, mask, ray_length


# speedup vs baseline: 1.5498x; 1.5498x over previous
import functools

import jax
import jax.numpy as jnp
from jax import lax
from jax.experimental import pallas as pl
from jax.experimental.pallas import tpu as pltpu
from jax.experimental.pallas import tpu_sc as plsc

_R = 8192
_S = 64
_C = 32
_P = _R * _S
_GAIN = 1.0
_NW = 32
_PW = _P // _NW
_CH = 128


_GDN = lax.GatherDimensionNumbers(offset_dims=(), collapsed_slice_dims=(0,),
                                  start_index_map=(0,))


def _lane_bcast(v, jv):
    return lax.gather(v, jv[:, None], _GDN, (1,),
                      mode=lax.GatherScatterMode.PROMISE_IN_BOUNDS)


def _compute_block(gb, wt_s, out_b):
    for g16 in range(_CH // 16):
        ws = []
        for t in range(3):
            tx = wt_s[2 * t, pl.ds(g16 * 16, 16)]
            ty = wt_s[2 * t + 1, pl.ds(g16 * 16, 16)]
            ws += [(1.0 - tx) * (1.0 - ty), tx * (1.0 - ty),
                   (1.0 - tx) * ty, tx * ty]

        hi_mask = jnp.full((16,), -65536, jnp.int32)
        rnd = jnp.full((16,), 32768, jnp.int32)

        def pt_body(jj, c2, g16=g16, ws=ws):
            j = g16 * 16 + jj
            jv = jnp.zeros((16,), jnp.int32) + jj
            wb = [_lane_bcast(w, jv) for w in ws]
            for g in range(2):
                acc_a = None
                acc_b = None
                for t in range(3):
                    for q in range(4):
                        v = gb[t][j, pl.ds(32 * q + 16 * g, 16)]
                        va = lax.bitcast_convert_type(v << 16, jnp.float32)
                        vb = lax.bitcast_convert_type(v & hi_mask, jnp.float32)
                        w = wb[4 * t + q]
                        if acc_a is None:
                            acc_a, acc_b = w * va, w * vb
                        else:
                            acc_a = acc_a + w * va
                            acc_b = acc_b + w * vb
                ia = lax.bitcast_convert_type(acc_a, jnp.int32) + rnd
                ib = lax.bitcast_convert_type(acc_b, jnp.int32) + rnd
                out_b[j, pl.ds(16 * g, 16)] = (
                    lax.shift_right_logical(ia, 16) | (ib & hi_mask))
            return c2

        lax.fori_loop(0, 16, pt_body, 0)


def _sc_gather(t_xy, t_yz, t_zx, idx_all, wt_all):
    mesh = plsc.VectorSubcoreMesh(core_axis_name="c", subcore_axis_name="s")
    nb = _PW // _CH

    @functools.partial(
        pl.kernel,
        mesh=mesh,
        out_type=jax.ShapeDtypeStruct((_P, 32), jnp.int32),
        scratch_types=[
            *[pltpu.VMEM((3, _CH), jnp.int32) for _ in range(2)],
            *[pltpu.VMEM((6, _CH), jnp.float32) for _ in range(2)],
            *[pltpu.VMEM((_CH, 128), jnp.int32) for _ in range(6)],
            pltpu.VMEM((_CH, 32), jnp.int32),
            *[pltpu.SemaphoreType.DMA for _ in range(5)],
        ],
    )
    def k(txy, tyz, tzx, idx_hbm, wt_hbm, out_hbm,
          i0, i1, w0, w1,
          ga0, ga1, ga2, gb0, gb1, gb2,
          outb, gsem0, gsem1, isem0, isem1, osem):
        tabs = (txy, tyz, tzx)
        idx_s = (i0, i1)
        wt_s = (w0, w1)
        gb = ((ga0, ga1, ga2), (gb0, gb1, gb2))
        gsem = (gsem0, gsem1)
        isem = (isem0, isem1)
        wid = lax.axis_index("s") * 2 + lax.axis_index("c")
        w_base = wid * _PW

        def fire_idx(b, p):
            pltpu.async_copy(idx_hbm.at[:, pl.ds(w_base + b * _CH, _CH)],
                             idx_s[p], isem[p])
            pltpu.async_copy(wt_hbm.at[:, pl.ds(w_base + b * _CH, _CH)],
                             wt_s[p], isem[p])

        def wait_idx(p):
            pltpu.make_async_copy(idx_hbm.at[:, pl.ds(0, _CH)],
                                  idx_s[p], isem[p]).wait()
            pltpu.make_async_copy(wt_hbm.at[:, pl.ds(0, _CH)],
                                  wt_s[p], isem[p]).wait()

        def fire_gathers(p):
            for t in range(3):
                pltpu.async_copy(tabs[t].at[idx_s[p].at[t]],
                                 gb[p][t], gsem[p])

        def wait_gathers(p):
            for t in range(3):
                pltpu.make_async_copy(tabs[t].at[idx_s[p].at[t]],
                                      gb[p][t], gsem[p]).wait()

        def wait_out():
            pltpu.make_async_copy(outb, out_hbm.at[pl.ds(w_base, _CH)],
                                  osem).wait()

        fire_idx(0, 0)
        wait_idx(0)
        fire_gathers(0)
        fire_idx(1, 1)

        def loop_body(bb, carry):
            for p in range(2):
                b = bb * 2 + p
                q = 1 - p
                wait_idx(q)
                fire_gathers(q)
                wait_gathers(p)
                if p == 0:
                    @pl.when(bb >= 1)
                    def _w():
                        wait_out()
                else:
                    wait_out()
                _compute_block(gb[p], wt_s[p], outb)
                pltpu.async_copy(
                    outb, out_hbm.at[pl.ds(w_base + b * _CH, _CH)], osem)
                bn = jnp.minimum(b + 2, nb - 1)
                fire_idx(bn, p)
            return carry

        lax.fori_loop(0, nb // 2, loop_body, 0)

        wait_idx(1)
        wait_gathers(0)
        wait_out()

    return k(t_xy, t_yz, t_zx, idx_all, wt_all)


def _softplus(x):
    return jnp.maximum(x, 0.0) + jnp.log1p(jnp.exp(-jnp.abs(x)))


def _tc_body(x_ref, enc_ref, nf_ref, wbe_ref, wbo_ref, w1t_ref, b0_ref, b1_ref,
             wop_ref, bop_ref, wct_ref, bc_ref, bg_ref,
             out_ref, nlt_ref, awhc_ref, aw_ref, awt_ref):
    s = pl.program_id(0)

    @pl.when(s == 0)
    def _init():
        nlt_ref[...] = jnp.zeros_like(nlt_ref)
        awhc_ref[...] = jnp.zeros_like(awhc_ref)
        aw_ref[...] = jnp.zeros_like(aw_ref)
        awt_ref[...] = jnp.zeros_like(awt_ref)

    v = x_ref[...]
    va = lax.bitcast_convert_type(v << 16, jnp.float32)
    vb = lax.bitcast_convert_type(
        v & jnp.int32(-65536), jnp.float32)
    z = (lax.dot_general(wbe_ref[...], va, (((1,), (1,)), ((), ())),
                         preferred_element_type=jnp.float32)
         + lax.dot_general(wbo_ref[...], vb, (((1,), (1,)), ((), ())),
                           preferred_element_type=jnp.float32))
    h = _softplus(z[0:_C, :] + b0_ref[...])
    h = _softplus(jnp.dot(w1t_ref[...], h,
                          preferred_element_type=jnp.float32) + b1_ref[...])
    sig_pre = jnp.sum(h * wop_ref[...], axis=0, keepdims=True) + bop_ref[...]
    sigma = _softplus(sig_pre)

    near = nf_ref[0:1, :]
    far = nf_ref[1:2, :]
    sf = (s.astype(jnp.float32) + 0.5) * (1.0 / _S)
    t = near + (far - near) * sf
    delta = (far - near) * (1.0 / _S)
    sd = sigma * delta

    nlt0 = nlt_ref[...]
    wgt = jnp.exp(-nlt0) * (1.0 - jnp.exp(-sd))
    nlt_ref[...] = nlt0 + sd

    hc = _softplus(h + z[_C:2 * _C, :] + enc_ref[...])
    awhc_ref[...] += wgt * hc
    aw_ref[...] += wgt
    awt_ref[...] += wgt * t

    @pl.when(s == _S - 1)
    def _fin():
        mask = 1.0 - jnp.exp(-nlt_ref[...])
        cf = (jnp.dot(wct_ref[...], awhc_ref[...],
                      preferred_element_type=jnp.float32)
              + bc_ref[...] * aw_ref[...])
        fr = cf[0:3, :] + (1.0 - mask) * bg_ref[...]
        out_ref[...] = jnp.concatenate(
            [fr, mask, awt_ref[...], jnp.zeros((3, _R), jnp.float32)], axis=0)


def _render(x, enc_t, nf, wbe, wbo, w1t, b0, b1, wop, bop, wct, bc, bg):
    full = lambda shape: pl.BlockSpec(shape, lambda s: (0, 0))
    return pl.pallas_call(
        _tc_body,
        grid=(_S,),
        in_specs=[
            pl.BlockSpec((_R, 32), lambda s: (s, 0)),
            full((_C, _R)), full((2, _R)), full((64, 32)), full((64, 32)),
            full((_C, _C)),
            full((_C, 1)), full((_C, 1)), full((_C, 1)), full((1, 1)),
            full((16, _C)), full((16, 1)), full((3, 1)),
        ],
        out_specs=full((8, _R)),
        out_shape=jax.ShapeDtypeStruct((8, _R), jnp.float32),
        scratch_shapes=[
            pltpu.VMEM((1, _R), jnp.float32),
            pltpu.VMEM((_C, _R), jnp.float32),
            pltpu.VMEM((1, _R), jnp.float32),
            pltpu.VMEM((1, _R), jnp.float32),
        ],
        compiler_params=pltpu.CompilerParams(
            dimension_semantics=("arbitrary",)),
    )(x, enc_t, nf, wbe, wbo, w1t, b0, b1, wop, bop, wct, bc, bg)


def _corner_idx(u, v):
    x = jnp.clip((u + 1.0) * (0.5 * 255.0), 0.0, 255.0)
    y = jnp.clip((v + 1.0) * (0.5 * 255.0), 0.0, 255.0)
    x0 = jnp.floor(x)
    y0 = jnp.floor(y)
    tx = x - x0
    ty = y - y0
    i00 = y0.astype(jnp.int32) * 256 + x0.astype(jnp.int32)
    return i00, tx, ty


def _shift_rows(a, n):
    return jnp.concatenate([a[n:], a[-n:]], axis=0)


def _pack_table(feat, color):
    base = jnp.concatenate([feat, color], axis=0).transpose(1, 2, 0).reshape(
        256 * 256, 64).astype(jnp.bfloat16)
    b32 = jax.lax.bitcast_convert_type(
        base.reshape(256 * 256, 32, 2), jnp.int32)
    s1 = _shift_rows(b32, 1)
    s256 = _shift_rows(b32, 256)
    s257 = _shift_rows(s1, 256)
    return jnp.concatenate([b32, s1, s256, s257], axis=1)


def kernel(rays, centers, rays_encoding, near, far, xy, yz, zx,
           xy_color, yz_color, zx_color, mlp_weights, mlp_biases,
           weight_opacity, bias_opacity, weight_color, bias_color, bg_color):
    sgrid = (jnp.arange(_S, dtype=jnp.float32) + 0.5) / _S
    t = near[None, :] + (far - near)[None, :] * sgrid[:, None]
    pts = centers[None, :, :] + t[..., None] * rays[None, :, :]
    px = pts[..., 0].reshape(_P)
    py = pts[..., 1].reshape(_P)
    pz = pts[..., 2].reshape(_P)

    idx_rows = []
    wt_rows = []
    for u, v in ((px, py), (py, pz), (pz, px)):
        i00, tx, ty = _corner_idx(u, v)
        idx_rows.append(i00)
        wt_rows += [tx, ty]
    idx_all = jnp.stack(idx_rows)
    wt_all = jnp.stack(wt_rows)

    t_xy = _pack_table(xy, xy_color)
    t_yz = _pack_table(yz, yz_color)
    t_zx = _pack_table(zx, zx_color)

    x_i32 = _sc_gather(t_xy, t_yz, t_zx, idx_all, wt_all)

    zero = jnp.zeros((_C, _C), jnp.float32)
    wb = jnp.concatenate([
        jnp.concatenate([_GAIN * mlp_weights[0].T, zero], axis=1),
        jnp.concatenate([zero, _GAIN * jnp.eye(_C, dtype=jnp.float32)], axis=1),
    ], axis=0)
    wbe = wb[:, 0::2]
    wbo = wb[:, 1::2]

    out = _render(
        x_i32, rays_encoding.T, jnp.stack([near, far]), wbe, wbo,
        mlp_weights[1].T, mlp_biases[0][:, None], mlp_biases[1][:, None],
        weight_opacity[:, None], bias_opacity.reshape(1, 1),
        weight_color.T, bias_color[:, None], bg_color[:, None])

    feature_render = out[0:3, :].T
    mask = out[3, :]
    ray_length = out[4, :]
    return feature_render, mask, ray_length


# speedup vs baseline: 1.7117x; 1.1044x over previous
import functools

import jax
import jax.numpy as jnp
from jax import lax
from jax.experimental import pallas as pl
from jax.experimental.pallas import tpu as pltpu
from jax.experimental.pallas import tpu_sc as plsc

_R = 8192
_S = 64
_C = 32
_P = _R * _S
_GAIN = 1.0
_NW = 32
_PW = _P // _NW
_CH = 128


_GDN = lax.GatherDimensionNumbers(offset_dims=(), collapsed_slice_dims=(0,),
                                  start_index_map=(0,))


def _lane_bcast(v, jv):
    return lax.gather(v, jv[:, None], _GDN, (1,),
                      mode=lax.GatherScatterMode.PROMISE_IN_BOUNDS)


def _compute_block(gb, wt_s, out_b):
    for g16 in range(_CH // 16):
        ws = []
        for t in range(3):
            tx = wt_s[2 * t, pl.ds(g16 * 16, 16)]
            ty = wt_s[2 * t + 1, pl.ds(g16 * 16, 16)]
            ws += [(1.0 - tx) * (1.0 - ty), tx * (1.0 - ty),
                   (1.0 - tx) * ty, tx * ty]

        hi_mask = jnp.full((16,), -65536, jnp.int32)
        rnd = jnp.full((16,), 32768, jnp.int32)

        def pt_body(jj, c2, g16=g16, ws=ws):
            j = g16 * 16 + jj
            jv = jnp.zeros((16,), jnp.int32) + jj
            wb = [_lane_bcast(w, jv) for w in ws]
            for g in range(2):
                acc_a = None
                acc_b = None
                for t in range(3):
                    for q in range(4):
                        v = gb[t][j, pl.ds(32 * q + 16 * g, 16)]
                        va = lax.bitcast_convert_type(v << 16, jnp.float32)
                        vb = lax.bitcast_convert_type(v & hi_mask, jnp.float32)
                        w = wb[4 * t + q]
                        if acc_a is None:
                            acc_a, acc_b = w * va, w * vb
                        else:
                            acc_a = acc_a + w * va
                            acc_b = acc_b + w * vb
                ia = lax.bitcast_convert_type(acc_a, jnp.int32) + rnd
                ib = lax.bitcast_convert_type(acc_b, jnp.int32) + rnd
                out_b[j, pl.ds(16 * g, 16)] = (
                    lax.shift_right_logical(ia, 16) | (ib & hi_mask))
            return c2

        lax.fori_loop(0, 16, pt_body, 0)


def _sc_gather(t_xy, t_yz, t_zx, idx_all, wt_all):
    mesh = plsc.VectorSubcoreMesh(core_axis_name="c", subcore_axis_name="s")
    nb = _PW // _CH

    @functools.partial(
        pl.kernel,
        mesh=mesh,
        out_type=jax.ShapeDtypeStruct((_P, 32), jnp.int32),
        scratch_types=[
            *[pltpu.VMEM((3, _CH), jnp.int32) for _ in range(2)],
            *[pltpu.VMEM((6, _CH), jnp.float32) for _ in range(2)],
            *[pltpu.VMEM((_CH, 128), jnp.int32) for _ in range(6)],
            pltpu.VMEM((_CH, 32), jnp.int32),
            *[pltpu.SemaphoreType.DMA for _ in range(5)],
        ],
    )
    def k(txy, tyz, tzx, idx_hbm, wt_hbm, out_hbm,
          i0, i1, w0, w1,
          ga0, ga1, ga2, gb0, gb1, gb2,
          outb, gsem0, gsem1, isem0, isem1, osem):
        tabs = (txy, tyz, tzx)
        idx_s = (i0, i1)
        wt_s = (w0, w1)
        gb = ((ga0, ga1, ga2), (gb0, gb1, gb2))
        gsem = (gsem0, gsem1)
        isem = (isem0, isem1)
        wid = lax.axis_index("s") * 2 + lax.axis_index("c")
        w_base = wid * _PW

        def fire_idx(b, p):
            pltpu.async_copy(idx_hbm.at[:, pl.ds(w_base + b * _CH, _CH)],
                             idx_s[p], isem[p])
            pltpu.async_copy(wt_hbm.at[:, pl.ds(w_base + b * _CH, _CH)],
                             wt_s[p], isem[p])

        def wait_idx(p):
            pltpu.make_async_copy(idx_hbm.at[:, pl.ds(0, _CH)],
                                  idx_s[p], isem[p]).wait()
            pltpu.make_async_copy(wt_hbm.at[:, pl.ds(0, _CH)],
                                  wt_s[p], isem[p]).wait()

        def fire_gathers(p):
            for t in range(3):
                pltpu.async_copy(tabs[t].at[idx_s[p].at[t]],
                                 gb[p][t], gsem[p])

        def wait_gathers(p):
            for t in range(3):
                pltpu.make_async_copy(tabs[t].at[idx_s[p].at[t]],
                                      gb[p][t], gsem[p]).wait()

        def wait_out():
            pltpu.make_async_copy(outb, out_hbm.at[pl.ds(w_base, _CH)],
                                  osem).wait()

        fire_idx(0, 0)
        wait_idx(0)
        fire_gathers(0)
        fire_idx(1, 1)

        def loop_body(bb, carry):
            for p in range(2):
                b = bb * 2 + p
                q = 1 - p
                wait_idx(q)
                fire_gathers(q)
                wait_gathers(p)
                if p == 0:
                    @pl.when(bb >= 1)
                    def _w():
                        wait_out()
                else:
                    wait_out()
                _compute_block(gb[p], wt_s[p], outb)
                pltpu.async_copy(
                    outb, out_hbm.at[pl.ds(w_base + b * _CH, _CH)], osem)
                bn = jnp.minimum(b + 2, nb - 1)
                fire_idx(bn, p)
            return carry

        lax.fori_loop(0, nb // 2, loop_body, 0)

        wait_idx(1)
        wait_gathers(0)
        wait_out()

    return k(t_xy, t_yz, t_zx, idx_all, wt_all)


def _softplus(x):
    return jnp.maximum(x, 0.0) + jnp.log1p(jnp.exp(-jnp.abs(x)))


def _tc_body(x_ref, enc_ref, nf_ref, wbe_ref, wbo_ref, w1t_ref, b0_ref, b1_ref,
             wop_ref, bop_ref, wct_ref, bc_ref, bg_ref,
             out_ref, nlt_ref, awhc_ref, aw_ref, awt_ref):
    s = pl.program_id(0)

    @pl.when(s == 0)
    def _init():
        nlt_ref[...] = jnp.zeros_like(nlt_ref)
        awhc_ref[...] = jnp.zeros_like(awhc_ref)
        aw_ref[...] = jnp.zeros_like(aw_ref)
        awt_ref[...] = jnp.zeros_like(awt_ref)

    v = x_ref[...]
    va = lax.bitcast_convert_type(v << 16, jnp.float32)
    vb = lax.bitcast_convert_type(
        v & jnp.int32(-65536), jnp.float32)
    z = (lax.dot_general(wbe_ref[...], va, (((1,), (1,)), ((), ())),
                         preferred_element_type=jnp.float32)
         + lax.dot_general(wbo_ref[...], vb, (((1,), (1,)), ((), ())),
                           preferred_element_type=jnp.float32))
    h = _softplus(z[0:_C, :] + b0_ref[...])
    h = _softplus(jnp.dot(w1t_ref[...], h,
                          preferred_element_type=jnp.float32) + b1_ref[...])
    sig_pre = jnp.sum(h * wop_ref[...], axis=0, keepdims=True) + bop_ref[...]
    sigma = _softplus(sig_pre)

    near = nf_ref[0:1, :]
    far = nf_ref[1:2, :]
    sf = (s.astype(jnp.float32) + 0.5) * (1.0 / _S)
    t = near + (far - near) * sf
    delta = (far - near) * (1.0 / _S)
    sd = sigma * delta

    nlt0 = nlt_ref[...]
    wgt = jnp.exp(-nlt0) * (1.0 - jnp.exp(-sd))
    nlt_ref[...] = nlt0 + sd

    hc = _softplus(h + z[_C:2 * _C, :] + enc_ref[...])
    awhc_ref[...] += wgt * hc
    aw_ref[...] += wgt
    awt_ref[...] += wgt * t

    @pl.when(s == _S - 1)
    def _fin():
        mask = 1.0 - jnp.exp(-nlt_ref[...])
        cf = (jnp.dot(wct_ref[...], awhc_ref[...],
                      preferred_element_type=jnp.float32)
              + bc_ref[...] * aw_ref[...])
        fr = cf[0:3, :] + (1.0 - mask) * bg_ref[...]
        out_ref[...] = jnp.concatenate(
            [fr, mask, awt_ref[...], jnp.zeros((3, _R), jnp.float32)], axis=0)


def _render(x, enc_t, nf, wbe, wbo, w1t, b0, b1, wop, bop, wct, bc, bg):
    full = lambda shape: pl.BlockSpec(shape, lambda s: (0, 0))
    return pl.pallas_call(
        _tc_body,
        grid=(_S,),
        in_specs=[
            pl.BlockSpec((_R, 32), lambda s: (s, 0)),
            full((_C, _R)), full((2, _R)), full((64, 32)), full((64, 32)),
            full((_C, _C)),
            full((_C, 1)), full((_C, 1)), full((_C, 1)), full((1, 1)),
            full((16, _C)), full((16, 1)), full((3, 1)),
        ],
        out_specs=full((8, _R)),
        out_shape=jax.ShapeDtypeStruct((8, _R), jnp.float32),
        scratch_shapes=[
            pltpu.VMEM((1, _R), jnp.float32),
            pltpu.VMEM((_C, _R), jnp.float32),
            pltpu.VMEM((1, _R), jnp.float32),
            pltpu.VMEM((1, _R), jnp.float32),
        ],
        compiler_params=pltpu.CompilerParams(
            dimension_semantics=("arbitrary",)),
    )(x, enc_t, nf, wbe, wbo, w1t, b0, b1, wop, bop, wct, bc, bg)


def _corner_idx(u, v):
    x = jnp.clip((u + 1.0) * (0.5 * 255.0), 0.0, 255.0)
    y = jnp.clip((v + 1.0) * (0.5 * 255.0), 0.0, 255.0)
    x0 = jnp.floor(x)
    y0 = jnp.floor(y)
    tx = x - x0
    ty = y - y0
    i00 = y0.astype(jnp.int32) * 256 + x0.astype(jnp.int32)
    return i00, tx, ty


def _shift_rows(a, n):
    return jnp.concatenate([a[n:], a[-n:]], axis=0)


def _prep_body(f_ref, c_ref, eye_ref, out_ref):
    ft = lax.dot_general(f_ref[...], eye_ref[...], (((0,), (0,)), ((), ())),
                         preferred_element_type=jnp.float32)
    ct = lax.dot_general(c_ref[...], eye_ref[...], (((0,), (0,)), ((), ())),
                         preferred_element_type=jnp.float32)
    rnd = jnp.int32(32768)
    fb = lax.bitcast_convert_type(ft, jnp.int32) + rnd
    cb = lax.bitcast_convert_type(ct, jnp.int32) + rnd
    out_ref[...] = (lax.shift_right_logical(fb, 16)
                    | (cb & jnp.int32(-65536)))


def _prep_pack(feat, color):
    nblk = 8
    n = 256 * 256 // nblk
    f2 = feat.reshape(_C, 256 * 256)
    c2 = color.reshape(_C, 256 * 256)
    eye = jnp.eye(_C, dtype=jnp.float32)
    return pl.pallas_call(
        _prep_body,
        grid=(nblk,),
        in_specs=[
            pl.BlockSpec((_C, n), lambda b: (0, b)),
            pl.BlockSpec((_C, n), lambda b: (0, b)),
            pl.BlockSpec((_C, _C), lambda b: (0, 0)),
        ],
        out_specs=pl.BlockSpec((n, _C), lambda b: (b, 0)),
        out_shape=jax.ShapeDtypeStruct((256 * 256, _C), jnp.int32),
    )(f2, c2, eye)


def _pack_table(feat, color):
    b32 = _prep_pack(feat, color)
    s1 = _shift_rows(b32, 1)
    s256 = _shift_rows(b32, 256)
    s257 = _shift_rows(s1, 256)
    return jnp.concatenate([b32, s1, s256, s257], axis=1)


def kernel(rays, centers, rays_encoding, near, far, xy, yz, zx,
           xy_color, yz_color, zx_color, mlp_weights, mlp_biases,
           weight_opacity, bias_opacity, weight_color, bias_color, bg_color):
    sgrid = (jnp.arange(_S, dtype=jnp.float32) + 0.5) / _S
    t = near[None, :] + (far - near)[None, :] * sgrid[:, None]
    px = (centers[:, 0][None, :] + t * rays[:, 0][None, :]).reshape(_P)
    py = (centers[:, 1][None, :] + t * rays[:, 1][None, :]).reshape(_P)
    pz = (centers[:, 2][None, :] + t * rays[:, 2][None, :]).reshape(_P)

    idx_rows = []
    wt_rows = []
    for u, v in ((px, py), (py, pz), (pz, px)):
        i00, tx, ty = _corner_idx(u, v)
        idx_rows.append(i00)
        wt_rows += [tx, ty]
    idx_all = jnp.stack(idx_rows)
    wt_all = jnp.stack(wt_rows)

    t_xy = _pack_table(xy, xy_color)
    t_yz = _pack_table(yz, yz_color)
    t_zx = _pack_table(zx, zx_color)

    x_i32 = _sc_gather(t_xy, t_yz, t_zx, idx_all, wt_all)

    zero = jnp.zeros((_C, _C), jnp.float32)
    wb = jnp.concatenate([
        jnp.concatenate([_GAIN * mlp_weights[0].T, zero], axis=1),
        jnp.concatenate([zero, _GAIN * jnp.eye(_C, dtype=jnp.float32)], axis=1),
    ], axis=0)
    wbe = wb[:, :_C]
    wbo = wb[:, _C:]

    out = _render(
        x_i32, rays_encoding.T, jnp.stack([near, far]), wbe, wbo,
        mlp_weights[1].T, mlp_biases[0][:, None], mlp_biases[1][:, None],
        weight_opacity[:, None], bias_opacity.reshape(1, 1),
        weight_color.T, bias_color[:, None], bg_color[:, None])

    feature_render = out[0:3, :].T
    mask = out[3, :]
    ray_length = out[4, :]
    return feature_render, mask, ray_length
